# baseline (speedup 1.0000x reference)
import functools

import jax
import jax.numpy as jnp
from jax import lax
from jax.experimental import pallas as pl
from jax.experimental.pallas import tpu as pltpu
from jax.experimental.pallas import tpu_sc as plsc

_INFO = plsc.get_sparse_core_info()
_NC, _NS = _INFO.num_cores, _INFO.num_subcores
_NW = _NC * _NS
_L = _INFO.num_lanes

_BLK = 512


@functools.partial(jax.jit, static_argnames=("n_rows", "n_types"))
def _sc_onehot(idx_flat, *, n_rows, n_types):
    b_per_w = n_rows // _NW
    n_groups = b_per_w // _L
    mesh = plsc.VectorSubcoreMesh(core_axis_name="c", subcore_axis_name="s")

    @functools.partial(
        pl.kernel,
        out_type=jax.ShapeDtypeStruct((n_types, n_rows), jnp.float32),
        mesh=mesh,
        scratch_types=[
            pltpu.VMEM((b_per_w,), jnp.int32),
            pltpu.VMEM((n_types, b_per_w), jnp.float32),
            pltpu.SemaphoreType.DMA,
        ],
    )
    def run(idx_hbm, oh_hbm, idx_v, oh_v, osem):
        wid = lax.axis_index("s") * _NC + lax.axis_index("c")
        base = wid * b_per_w
        pltpu.sync_copy(idx_hbm.at[pl.ds(base, b_per_w)], idx_v)

        @pl.loop(0, n_groups)
        def _grp(g):
            tvec = idx_v[pl.ds(g * _L, _L)]
            for r in range(n_types):
                oh_v[r, pl.ds(g * _L, _L)] = jnp.where(
                    tvec == r, jnp.float32(1.0), jnp.float32(0.0))

        for r in range(n_types):
            pltpu.async_copy(
                oh_v.at[r], oh_hbm.at[r, pl.ds(base, b_per_w)], osem)
        for r in range(n_types):
            pltpu.make_async_copy(
                oh_v.at[r], oh_hbm.at[r, pl.ds(base, b_per_w)], osem).wait()

    return run(idx_flat)


@functools.partial(jax.jit, static_argnames=("n_rows", "d_model", "n_types"))
def _tc_expand(ohT, w3, *, n_rows, d_model, n_types):
    grid = n_rows // _BLK

    def body(oh_ref, w3_ref, o_ref):
        ohb = oh_ref[...].astype(jnp.bfloat16)
        oh3 = jnp.concatenate([ohb, ohb, ohb], axis=0)
        o_ref[...] = lax.dot_general(
            oh3, w3_ref[...], (((0,), (0,)), ((), ())),
            preferred_element_type=jnp.float32)

    return pl.pallas_call(
        body,
        grid=(grid,),
        in_specs=[
            pl.BlockSpec((n_types, _BLK), lambda i: (0, i)),
            pl.BlockSpec((3 * n_types, d_model), lambda i: (0, 0)),
        ],
        out_specs=pl.BlockSpec((_BLK, d_model), lambda i: (i, 0)),
        out_shape=jax.ShapeDtypeStruct((n_rows, d_model), jnp.float32),
    )(ohT, w3)


def kernel(token_types, weight):
    n_rows = token_types.size
    n_types, d_model = weight.shape
    idx_flat = token_types.reshape(-1).astype(jnp.int32)
    h = weight.astype(jnp.bfloat16)
    r1 = weight - h.astype(jnp.float32)
    m = r1.astype(jnp.bfloat16)
    l = (r1 - m.astype(jnp.float32)).astype(jnp.bfloat16)
    w3 = jnp.concatenate([h, m, l], axis=0)
    ohT = _sc_onehot(idx_flat, n_rows=n_rows, n_types=n_types)
    out = _tc_expand(ohT, w3, n_rows=n_rows, d_model=d_model,
                     n_types=n_types)
    return out.reshape(token_types.shape + (d_model,))
